# baseline (speedup 1.0000x reference)
import functools

import jax
import jax.numpy as jnp
from jax import lax
from jax.experimental import pallas as pl
from jax.experimental.pallas import tpu as pltpu, tpu_sc as plsc


@functools.cache
def _make_gather(batch, hist, v, d):
    info = plsc.get_sparse_core_info()
    nc, ns = info.num_cores, info.num_subcores
    nw = nc * ns
    bc = batch // nw
    nct = bc // 128
    dt = d // 8
    ntask = hist * nct
    assert bc % 128 == 0 and d % 8 == 0 and ntask % 4 == 0 and v % 4 == 0

    mesh = plsc.VectorSubcoreMesh(core_axis_name="c", subcore_axis_name="s")

    @functools.partial(
        pl.kernel,
        mesh=mesh,
        compiler_params=pltpu.CompilerParams(
            use_tc_tiling_on_sc=False, needs_layout_passes=False),
        out_type=jax.ShapeDtypeStruct((hist, dt, batch // 128, 8, 128),
                                      jnp.float32),
        scratch_types=[
            pltpu.VMEM((hist, nct, 128), jnp.int32),
            pltpu.VMEM((4, 128), jnp.int32),
            pltpu.VMEM((128, 4 * d), jnp.float32),
            pltpu.VMEM((128, 4 * d), jnp.float32),
            pltpu.VMEM((128, 4 * d), jnp.float32),
            pltpu.VMEM((128, 4 * d), jnp.float32),
            pltpu.VMEM((dt, 1, 8, 128), jnp.float32),
            pltpu.VMEM((dt, 1, 8, 128), jnp.float32),
            pltpu.SemaphoreType.DMA,
            pltpu.SemaphoreType.DMA,
            pltpu.SemaphoreType.DMA,
            pltpu.SemaphoreType.DMA,
            pltpu.SemaphoreType.DMA,
            pltpu.SemaphoreType.DMA,
        ],
    )
    def gather(table_hbm, x3_hbm, out_hbm, idx_v, idx4_v, r0, r1, r2, r3,
               tr0, tr1, s0, s1, s2, s3, sf0, sf1):
        wid = lax.axis_index("s") * nc + lax.axis_index("c")
        rows = (r0, r1, r2, r3)
        sems = (s0, s1, s2, s3)
        trs = (tr0, tr1)
        sfs = (sf0, sf1)

        pltpu.sync_copy(x3_hbm.at[:, pl.ds(wid * nct, nct), :], idx_v)

        viota = lax.iota(jnp.int32, 16)

        def fire(t, k):
            h, j = t // nct, t % nct
            for bl in range(8):
                iv = idx_v[h, j, pl.ds(bl * 16, 16)]
                idx4_v[k, pl.ds(bl * 16, 16)] = jnp.right_shift(iv, 2)
            pltpu.async_copy(table_hbm.at[idx4_v.at[k]], rows[k], sems[k])

        def drain(k):
            pltpu.make_async_copy(
                table_hbm.at[pl.ds(0, 128)], rows[k], sems[k]).wait()

        def out_slab(t):
            h, j = t // nct, t % nct
            return out_hbm.at[h, :, pl.ds(wid * nct + j, 1), :, :]

        def trans(t, k, tr):
            h, j = t // nct, t % nct
            r = rows[k]

            def blk_body(blk, carry):
                off = blk * 16
                ridx = viota + off
                islice = idx_v[h, j, pl.ds(off, 16)]
                qcol = jnp.left_shift(
                    jnp.bitwise_and(islice, jnp.int32(3)), 5)
                v0 = plsc.load_gather(r, [ridx, qcol])
                v1 = plsc.load_gather(r, [ridx, qcol + 1])
                for dd in range(2, d):
                    nxt = plsc.load_gather(r, [ridx, qcol + dd])
                    tr[(dd - 2) // 8, 0, (dd - 2) % 8, pl.ds(off, 16)] = v0
                    v0, v1 = v1, nxt
                tr[(d - 2) // 8, 0, (d - 2) % 8, pl.ds(off, 16)] = v0
                tr[(d - 1) // 8, 0, (d - 1) % 8, pl.ds(off, 16)] = v1
                return carry

            lax.fori_loop(0, 8, blk_body, 0)

        fire(0, 0)
        fire(1, 1)
        fire(2, 2)

        def quad(q, carry):
            for i in range(4):
                t = 4 * q + i

                @pl.when(t + 3 < ntask)
                def _():
                    fire(t + 3, (i + 3) % 4)

                drain(i)

                @pl.when(t >= 2)
                def _():
                    pltpu.make_async_copy(
                        trs[i % 2], out_slab(t - 2), sfs[i % 2]).wait()

                trans(t, i, trs[i % 2])
                pltpu.async_copy(trs[i % 2], out_slab(t), sfs[i % 2])
            return carry

        lax.fori_loop(0, ntask // 4, quad, 0)
        pltpu.make_async_copy(tr0, out_slab(ntask - 2), sf0).wait()
        pltpu.make_async_copy(tr1, out_slab(ntask - 1), sf1).wait()

    return gather


def kernel(x, weight):
    b, h = x.shape
    v, d = weight.shape
    x3 = x.T.reshape(h, b // 128, 128).astype(jnp.int32)
    w4 = weight.reshape(v // 4, 4 * d)
    out5 = _make_gather(b, h, v, d)(w4, x3)
    return out5.transpose(2, 4, 0, 1, 3).reshape(b, h, d)
